# baseline (speedup 1.0000x reference)
import jax
import jax.numpy as jnp
from jax.experimental import pallas as pl


def _matmul_kernel(a_ref, b_ref, o_ref):
    o_ref[...] = jnp.dot(a_ref[...], b_ref[...],
                         preferred_element_type=jnp.float32)


def _agg1_kernel(adj_ref, p_ref, in_ref, kg_ref, bg_ref, w2_ref,
                 out_ref, p2_ref):
    t = jnp.dot(adj_ref[...], p_ref[...], preferred_element_type=jnp.float32)
    t = jnp.maximum(t, 0.0)
    g = jax.nn.sigmoid(
        jnp.dot(in_ref[...], kg_ref[...], preferred_element_type=jnp.float32)
        + bg_ref[...])
    h = g * t + (1.0 - g) * in_ref[...]
    out_ref[...] = h
    p2_ref[...] = jnp.dot(h, w2_ref[...], preferred_element_type=jnp.float32)


def _agg2_kernel(adj_ref, p_ref, in_ref, kg_ref, bg_ref, out_ref):
    t = jnp.dot(adj_ref[...], p_ref[...], preferred_element_type=jnp.float32)
    t = jnp.maximum(t, 0.0)
    g = jax.nn.sigmoid(
        jnp.dot(in_ref[...], kg_ref[...], preferred_element_type=jnp.float32)
        + bg_ref[...])
    out_ref[...] = g * t + (1.0 - g) * in_ref[...]


def kernel(x, adj, kernel_gate, bias_gate, Weight_1, Weight_2):
    n, d = x.shape
    bg = bias_gate.reshape(1, d)
    bm = next(b for b in (200, 80, 40, 16, 8, n) if n % b == 0)
    grid = (n // bm,)

    nd = jax.ShapeDtypeStruct((n, d), jnp.float32)
    row_spec = pl.BlockSpec((bm, d), lambda i: (i, 0))
    full_spec = pl.BlockSpec((n, d), lambda i: (0, 0))
    sq_spec = pl.BlockSpec((d, d), lambda i: (0, 0))
    bias_spec = pl.BlockSpec((1, d), lambda i: (0, 0))
    adj_spec = pl.BlockSpec((bm, n), lambda i: (i, 0))

    p1 = pl.pallas_call(
        _matmul_kernel,
        grid=grid,
        in_specs=[row_spec, sq_spec],
        out_specs=row_spec,
        out_shape=nd,
    )(x, Weight_1)

    hg1, p2 = pl.pallas_call(
        _agg1_kernel,
        grid=grid,
        in_specs=[adj_spec, full_spec, row_spec, sq_spec, bias_spec, sq_spec],
        out_specs=[row_spec, row_spec],
        out_shape=[nd, nd],
    )(adj, p1, x, kernel_gate, bg, Weight_2)

    out = pl.pallas_call(
        _agg2_kernel,
        grid=grid,
        in_specs=[adj_spec, full_spec, row_spec, sq_spec, bias_spec],
        out_specs=row_spec,
        out_shape=nd,
    )(adj, p2, hg1, kernel_gate, bg)
    return out


# speedup vs baseline: 1.0451x; 1.0451x over previous
import jax
import jax.numpy as jnp
from jax.experimental import pallas as pl


def _matmul_kernel(a_ref, b_ref, o_ref):
    o_ref[...] = jnp.dot(a_ref[...], b_ref[...],
                         preferred_element_type=jnp.float32)


def _agg1_kernel(adj_ref, p_ref, in_ref, kg_ref, bg_ref, w2_ref,
                 out_ref, p2_ref):
    t = jnp.dot(adj_ref[...], p_ref[...], preferred_element_type=jnp.float32)
    t = jnp.maximum(t, 0.0)
    g = jax.nn.sigmoid(
        jnp.dot(in_ref[...], kg_ref[...], preferred_element_type=jnp.float32)
        + bg_ref[...])
    h = g * t + (1.0 - g) * in_ref[...]
    out_ref[...] = h
    p2_ref[...] = jnp.dot(h, w2_ref[...], preferred_element_type=jnp.float32)


def _agg2_kernel(adj_ref, p_ref, in_ref, kg_ref, bg_ref, out_ref):
    t = jnp.dot(adj_ref[...], p_ref[...], preferred_element_type=jnp.float32)
    t = jnp.maximum(t, 0.0)
    g = jax.nn.sigmoid(
        jnp.dot(in_ref[...], kg_ref[...], preferred_element_type=jnp.float32)
        + bg_ref[...])
    out_ref[...] = g * t + (1.0 - g) * in_ref[...]


def kernel(x, adj, kernel_gate, bias_gate, Weight_1, Weight_2):
    n, d = x.shape
    bg = bias_gate.reshape(1, d)
    bm = next(b for b in (400, 200, 80, 40, 16, 8, n) if n % b == 0)
    grid = (n // bm,)

    nd = jax.ShapeDtypeStruct((n, d), jnp.float32)
    row_spec = pl.BlockSpec((bm, d), lambda i: (i, 0))
    full_spec = pl.BlockSpec((n, d), lambda i: (0, 0))
    sq_spec = pl.BlockSpec((d, d), lambda i: (0, 0))
    bias_spec = pl.BlockSpec((1, d), lambda i: (0, 0))
    adj_spec = pl.BlockSpec((bm, n), lambda i: (i, 0))

    p1 = pl.pallas_call(
        _matmul_kernel,
        grid=grid,
        in_specs=[row_spec, sq_spec],
        out_specs=row_spec,
        out_shape=nd,
    )(x, Weight_1)

    hg1, p2 = pl.pallas_call(
        _agg1_kernel,
        grid=grid,
        in_specs=[adj_spec, full_spec, row_spec, sq_spec, bias_spec, sq_spec],
        out_specs=[row_spec, row_spec],
        out_shape=[nd, nd],
    )(adj, p1, x, kernel_gate, bg, Weight_2)

    out = pl.pallas_call(
        _agg2_kernel,
        grid=grid,
        in_specs=[adj_spec, full_spec, row_spec, sq_spec, bias_spec],
        out_specs=row_spec,
        out_shape=nd,
    )(adj, p2, hg1, kernel_gate, bg)
    return out


# speedup vs baseline: 1.1152x; 1.0670x over previous
import jax
import jax.numpy as jnp
from jax.experimental import pallas as pl


def _layer_kernel(adj_ref, full_ref, blk_ref, kg_ref, bg_ref, w_ref, out_ref):
    a = jnp.dot(adj_ref[...], full_ref[...], preferred_element_type=jnp.float32)
    t = jnp.maximum(
        jnp.dot(a, w_ref[...], preferred_element_type=jnp.float32), 0.0)
    g = jax.nn.sigmoid(
        jnp.dot(blk_ref[...], kg_ref[...], preferred_element_type=jnp.float32)
        + bg_ref[...])
    out_ref[...] = g * t + (1.0 - g) * blk_ref[...]


def kernel(x, adj, kernel_gate, bias_gate, Weight_1, Weight_2):
    n, d = x.shape
    bg = bias_gate.reshape(1, d)
    bm = next(b for b in (400, 200, 80, 40, 16, 8, n) if n % b == 0)
    grid = (n // bm,)

    nd = jax.ShapeDtypeStruct((n, d), jnp.float32)
    row_spec = pl.BlockSpec((bm, d), lambda i: (i, 0))
    full_spec = pl.BlockSpec((n, d), lambda i: (0, 0))
    sq_spec = pl.BlockSpec((d, d), lambda i: (0, 0))
    bias_spec = pl.BlockSpec((1, d), lambda i: (0, 0))
    adj_spec = pl.BlockSpec((bm, n), lambda i: (i, 0))

    layer = pl.pallas_call(
        _layer_kernel,
        grid=grid,
        in_specs=[adj_spec, full_spec, row_spec, sq_spec, bias_spec, sq_spec],
        out_specs=row_spec,
        out_shape=nd,
    )

    hg1 = layer(adj, x, x, kernel_gate, bg, Weight_1)
    return layer(adj, hg1, hg1, kernel_gate, bg, Weight_2)
